# baseline (device time: 23882 ns/iter reference)
import jax
import jax.numpy as jnp
from jax import lax
from jax.experimental import pallas as pl
from jax.experimental.pallas import tpu as pltpu

N_DEV = 8
MASKS = (1, 3, 4)

_BANDS = (
    (0, 384, (0, 1, 2)),
    (384, 384, (1, 2, 0)),
    (768, 256, (2, 0, 1)),
)
_RSPLIT = 16
_RROWS = 1024 // _RSPLIT

PARTS = tuple(
    (r * _RROWS, _RROWS, cbase, clen, order, r)
    for (cbase, clen, order) in _BANDS
    for r in range(_RSPLIT)
)
_ISSUE = tuple(b * _RSPLIT + r for r in range(_RSPLIT) for b in (0, 1, 2))
_ORDER = tuple(b * _RSPLIT + r for r in range(_RSPLIT) for b in (1, 2, 0))

_RB_STEP = {0: 0, 1: _RROWS >> 1, 2: (_RROWS >> 1) + (_RROWS >> 2)}
_RB_SLOT = (_RROWS >> 1) + (_RROWS >> 2) + (_RROWS >> 2)
_RB_ROWS = _RSPLIT * _RB_SLOT


def kernel(x):
    _, m, n = x.shape
    n_parts = len(PARTS)

    def body(x_ref, out_ref, work_ref, rb_ref, send_sems, recv_sems):
        p = lax.axis_index("i")
        b = [(p ^ (p >> 1)) & 1, (p >> 1) & 1, (p >> 2) & 1]

        offs = [jnp.int32(base) for (base, _, _, _, _, _) in PARTS]
        pending = [None] * n_parts

        barrier_sem = pltpu.get_barrier_semaphore()
        for mask in MASKS:
            pl.semaphore_signal(
                barrier_sem, inc=1,
                device_id=(p ^ mask,), device_id_type=pl.DeviceIdType.MESH,
            )
        for pi in _ISSUE:
            rbase, rlen, cbase, clen, order, _ = PARTS[pi]
            half = rlen >> 1
            send_off = rbase + (1 - b[order[0]]) * half
            cs = pl.ds(cbase, clen)
            work_ref[pl.ds(send_off, half), cs] = x_ref[
                0, pl.ds(send_off, half), cs
            ].astype(jnp.bfloat16)
        pl.semaphore_wait(barrier_sem, len(MASKS))

        def start_rs(pi, s):
            rbase, rlen, cbase, clen, order, slot = PARTS[pi]
            half = rlen >> (s + 1)
            dim = order[s]
            keep_off = offs[pi] + b[dim] * half
            send_off = offs[pi] + (1 - b[dim]) * half
            rb_off = slot * _RB_SLOT + _RB_STEP[s]
            cs = pl.ds(cbase, clen)
            rdma = pltpu.make_async_remote_copy(
                src_ref=work_ref.at[pl.ds(send_off, half), cs],
                dst_ref=rb_ref.at[pl.ds(rb_off, half), cs],
                send_sem=send_sems.at[pi],
                recv_sem=recv_sems.at[pi],
                device_id=(p ^ MASKS[dim],),
                device_id_type=pl.DeviceIdType.MESH,
            )
            rdma.start()
            offs[pi] = keep_off
            pending[pi] = (rdma, keep_off, half, rb_off)

        def finish_rs(pi, into_out=False):
            rdma, keep_off, half, rb_off = pending[pi]
            rdma.wait()
            _, _, cbase, clen, _, _ = PARTS[pi]
            cs = pl.ds(cbase, clen)
            dst = out_ref if into_out else work_ref
            dst[pl.ds(keep_off, half), cs] = (
                work_ref[pl.ds(keep_off, half), cs]
                + rb_ref[pl.ds(rb_off, half), cs]
            )

        def start_ar2(pi):
            rbase, rlen, cbase, clen, order, slot = PARTS[pi]
            cur = rlen >> 2
            dim = order[2]
            rb_off = slot * _RB_SLOT + _RB_STEP[2]
            cs = pl.ds(cbase, clen)
            rdma = pltpu.make_async_remote_copy(
                src_ref=work_ref.at[pl.ds(offs[pi], cur), cs],
                dst_ref=rb_ref.at[pl.ds(rb_off, cur), cs],
                send_sem=send_sems.at[pi],
                recv_sem=recv_sems.at[pi],
                device_id=(p ^ MASKS[dim],),
                device_id_type=pl.DeviceIdType.MESH,
            )
            rdma.start()
            pending[pi] = (rdma, offs[pi], cur, rb_off)

        def finish_ar2(pi):
            rdma, off, cur, rb_off = pending[pi]
            rdma.wait()
            _, _, cbase, clen, _, _ = PARTS[pi]
            cs = pl.ds(cbase, clen)
            out_ref[pl.ds(off, cur), cs] = (
                work_ref[pl.ds(off, cur), cs]
                + rb_ref[pl.ds(rb_off, cur), cs]
            )

        def start_ag(pi, s):
            rbase, rlen, cbase, clen, order, slot = PARTS[pi]
            cur = rlen >> (s + 1)
            dim = order[s]
            cs = pl.ds(cbase, clen)
            rdma = pltpu.make_async_remote_copy(
                src_ref=out_ref.at[pl.ds(offs[pi], cur), cs],
                dst_ref=out_ref.at[pl.ds(offs[pi], cur), cs],
                send_sem=send_sems.at[pi],
                recv_sem=recv_sems.at[pi],
                device_id=(p ^ MASKS[dim],),
                device_id_type=pl.DeviceIdType.MESH,
            )
            rdma.start()
            offs[pi] = offs[pi] - b[dim] * cur
            pending[pi] = (rdma,)

        for pi in _ISSUE:
            start_rs(pi, 0)
        for pi in _ISSUE:
            rbase, rlen, cbase, clen, _, _ = PARTS[pi]
            half = rlen >> 1
            cs = pl.ds(cbase, clen)
            work_ref[pl.ds(offs[pi], half), cs] = x_ref[
                0, pl.ds(offs[pi], half), cs
            ].astype(jnp.bfloat16)
        for pi in _ORDER:
            finish_rs(pi)
            start_rs(pi, 1)
        for pi in _ORDER:
            finish_rs(pi)
            start_ar2(pi)
        for pi in _ORDER:
            finish_ar2(pi)
            start_ag(pi, 1)
        for pi in _ORDER:
            pending[pi][0].wait()
            start_ag(pi, 0)
        for pi in _ORDER:
            pending[pi][0].wait()

    return pl.pallas_call(
        body,
        out_shape=jax.ShapeDtypeStruct((m, n), jnp.bfloat16),
        in_specs=[pl.BlockSpec(memory_space=pltpu.VMEM)],
        out_specs=pl.BlockSpec(memory_space=pltpu.VMEM),
        scratch_shapes=[
            pltpu.VMEM((m, n), jnp.bfloat16),
            pltpu.VMEM((_RB_ROWS, n), jnp.bfloat16),
            pltpu.SemaphoreType.DMA((len(PARTS),)),
            pltpu.SemaphoreType.DMA((len(PARTS),)),
        ],
        compiler_params=pltpu.CompilerParams(collective_id=0),
    )(x)


# device time: 23812 ns/iter; 1.0029x vs baseline; 1.0029x over previous
import jax
import jax.numpy as jnp
from jax import lax
from jax.experimental import pallas as pl
from jax.experimental.pallas import tpu as pltpu

N_DEV = 8
MASKS = (1, 3, 4)

_BANDS = (
    (0, 384, (0, 1, 2)),
    (384, 384, (1, 2, 0)),
    (768, 256, (2, 0, 1)),
)
_RSPLIT = 8
_RROWS = 1024 // _RSPLIT

PARTS = tuple(
    (r * _RROWS, _RROWS, cbase, clen, order, r)
    for (cbase, clen, order) in _BANDS
    for r in range(_RSPLIT)
)
_ISSUE = tuple(b * _RSPLIT + r for r in range(_RSPLIT) for b in (0, 1, 2))
_ORDER = tuple(b * _RSPLIT + r for r in range(_RSPLIT) for b in (1, 2, 0))

_RB_STEP = {0: 0, 1: _RROWS >> 1, 2: (_RROWS >> 1) + (_RROWS >> 2)}
_RB_SLOT = (_RROWS >> 1) + (_RROWS >> 2) + (_RROWS >> 2)
_RB_ROWS = _RSPLIT * _RB_SLOT


def kernel(x):
    _, m, n = x.shape
    n_parts = len(PARTS)

    def body(x_ref, out_ref, work_ref, rb_ref, send_sems, recv_sems):
        p = lax.axis_index("i")
        b = [(p ^ (p >> 1)) & 1, (p >> 1) & 1, (p >> 2) & 1]

        offs = [jnp.int32(base) for (base, _, _, _, _, _) in PARTS]
        pending = [None] * n_parts

        barrier_sem = pltpu.get_barrier_semaphore()
        for mask in MASKS:
            pl.semaphore_signal(
                barrier_sem, inc=1,
                device_id=(p ^ mask,), device_id_type=pl.DeviceIdType.MESH,
            )
        for pi in _ISSUE:
            rbase, rlen, cbase, clen, order, _ = PARTS[pi]
            half = rlen >> 1
            send_off = rbase + (1 - b[order[0]]) * half
            cs = pl.ds(cbase, clen)
            work_ref[pl.ds(send_off, half), cs] = x_ref[
                0, pl.ds(send_off, half), cs
            ].astype(jnp.bfloat16)
        pl.semaphore_wait(barrier_sem, len(MASKS))

        def start_rs(pi, s):
            rbase, rlen, cbase, clen, order, slot = PARTS[pi]
            half = rlen >> (s + 1)
            dim = order[s]
            keep_off = offs[pi] + b[dim] * half
            send_off = offs[pi] + (1 - b[dim]) * half
            rb_off = slot * _RB_SLOT + _RB_STEP[s]
            cs = pl.ds(cbase, clen)
            rdma = pltpu.make_async_remote_copy(
                src_ref=work_ref.at[pl.ds(send_off, half), cs],
                dst_ref=rb_ref.at[pl.ds(rb_off, half), cs],
                send_sem=send_sems.at[pi],
                recv_sem=recv_sems.at[pi],
                device_id=(p ^ MASKS[dim],),
                device_id_type=pl.DeviceIdType.MESH,
            )
            rdma.start()
            offs[pi] = keep_off
            pending[pi] = (rdma, keep_off, half, rb_off)

        def finish_rs(pi, into_out=False):
            rdma, keep_off, half, rb_off = pending[pi]
            rdma.wait()
            _, _, cbase, clen, _, _ = PARTS[pi]
            cs = pl.ds(cbase, clen)
            dst = out_ref if into_out else work_ref
            dst[pl.ds(keep_off, half), cs] = (
                work_ref[pl.ds(keep_off, half), cs]
                + rb_ref[pl.ds(rb_off, half), cs]
            )

        def start_ar2(pi):
            rbase, rlen, cbase, clen, order, slot = PARTS[pi]
            cur = rlen >> 2
            dim = order[2]
            rb_off = slot * _RB_SLOT + _RB_STEP[2]
            cs = pl.ds(cbase, clen)
            rdma = pltpu.make_async_remote_copy(
                src_ref=work_ref.at[pl.ds(offs[pi], cur), cs],
                dst_ref=rb_ref.at[pl.ds(rb_off, cur), cs],
                send_sem=send_sems.at[pi],
                recv_sem=recv_sems.at[pi],
                device_id=(p ^ MASKS[dim],),
                device_id_type=pl.DeviceIdType.MESH,
            )
            rdma.start()
            pending[pi] = (rdma, offs[pi], cur, rb_off)

        def finish_ar2(pi):
            rdma, off, cur, rb_off = pending[pi]
            rdma.wait()
            _, _, cbase, clen, _, _ = PARTS[pi]
            cs = pl.ds(cbase, clen)
            out_ref[pl.ds(off, cur), cs] = (
                work_ref[pl.ds(off, cur), cs]
                + rb_ref[pl.ds(rb_off, cur), cs]
            )

        def start_ag(pi, s):
            rbase, rlen, cbase, clen, order, slot = PARTS[pi]
            cur = rlen >> (s + 1)
            dim = order[s]
            cs = pl.ds(cbase, clen)
            rdma = pltpu.make_async_remote_copy(
                src_ref=out_ref.at[pl.ds(offs[pi], cur), cs],
                dst_ref=out_ref.at[pl.ds(offs[pi], cur), cs],
                send_sem=send_sems.at[pi],
                recv_sem=recv_sems.at[pi],
                device_id=(p ^ MASKS[dim],),
                device_id_type=pl.DeviceIdType.MESH,
            )
            rdma.start()
            offs[pi] = offs[pi] - b[dim] * cur
            pending[pi] = (rdma,)

        for pi in _ISSUE:
            start_rs(pi, 0)
        for pi in _ISSUE:
            rbase, rlen, cbase, clen, _, _ = PARTS[pi]
            half = rlen >> 1
            cs = pl.ds(cbase, clen)
            work_ref[pl.ds(offs[pi], half), cs] = x_ref[
                0, pl.ds(offs[pi], half), cs
            ].astype(jnp.bfloat16)
        for pi in _ORDER:
            finish_rs(pi)
            start_rs(pi, 1)
        for pi in _ORDER:
            finish_rs(pi)
            start_ar2(pi)
        for pi in _ORDER:
            finish_ar2(pi)
            start_ag(pi, 1)
        for pi in _ORDER:
            pending[pi][0].wait()
            start_ag(pi, 0)
        for pi in _ORDER:
            pending[pi][0].wait()

    return pl.pallas_call(
        body,
        out_shape=jax.ShapeDtypeStruct((m, n), jnp.bfloat16),
        in_specs=[pl.BlockSpec(memory_space=pltpu.VMEM)],
        out_specs=pl.BlockSpec(memory_space=pltpu.VMEM),
        scratch_shapes=[
            pltpu.VMEM((m, n), jnp.bfloat16),
            pltpu.VMEM((_RB_ROWS, n), jnp.bfloat16),
            pltpu.SemaphoreType.DMA((len(PARTS),)),
            pltpu.SemaphoreType.DMA((len(PARTS),)),
        ],
        compiler_params=pltpu.CompilerParams(collective_id=0),
    )(x)
